# baseline (speedup 1.0000x reference)
import jax
import jax.numpy as jnp
from jax import lax
from jax.experimental import pallas as pl
from jax.experimental.pallas import tpu as pltpu
from jax.experimental.pallas import tpu_sc as plsc

N = 10000
D = 128
BETA = 0.5
NC = 2
NS = 16
NW = NC * NS
ROW = 128
ICH = 8
NP = 10240
ZCH = NP // NS
BLK = 1000
GRID = N // BLK


def _sc_agg_body(with_deg, m_hbm, src_hbm, dst_hbm, zacc_hbm, *rest):
    if with_deg:
        (zdeg_hbm, agg_out, deg_out, acc_sh, deg_sh, srcb, dstb,
         rows0, rows1, ones_v, gsem0, gsem1, dsem) = rest
    else:
        zdeg_hbm = deg_out = deg_sh = ones_v = dsem = None
        (agg_out, acc_sh, srcb, dstb, rows0, rows1, gsem0, gsem1) = rest

    c = lax.axis_index("c")
    s = lax.axis_index("s")
    w = c * NS + s
    nrows = src_hbm.shape[0] // NW
    base = w * nrows

    pltpu.sync_copy(zacc_hbm.at[pl.ds(s * ZCH, ZCH)],
                    acc_sh.at[pl.ds(s * ZCH, ZCH)])
    if with_deg:
        pltpu.sync_copy(zdeg_hbm.at[pl.ds(s * ZCH, ZCH)],
                        deg_sh.at[pl.ds(s * ZCH, ZCH)])
        for k in range(ROW // 16):
            ones_v[pl.ds(k * 16, 16)] = jnp.ones((16,), jnp.float32)
    plsc.subcore_barrier()

    def chunk_body(ci, carry):
        rb = base + ci * ICH
        pltpu.sync_copy(src_hbm.at[pl.ds(rb, ICH)], srcb)
        pltpu.sync_copy(dst_hbm.at[pl.ds(rb, ICH)], dstb)
        pltpu.async_copy(m_hbm.at[srcb.at[0]], rows0, gsem0)
        pltpu.async_copy(m_hbm.at[srcb.at[1]], rows1, gsem1)
        for j in range(ICH):
            buf, sem = (rows0, gsem0) if j % 2 == 0 else (rows1, gsem1)
            pltpu.make_async_copy(m_hbm.at[srcb.at[j]], buf, sem).wait()
            pltpu.sync_copy(buf, acc_sh.at[dstb.at[j]], add=True)
            if with_deg:
                pltpu.async_copy(ones_v, deg_sh.at[dstb.at[j]], dsem,
                                 add=True)
            if j + 2 < ICH:
                pltpu.async_copy(m_hbm.at[srcb.at[j + 2]], buf, sem)
        if with_deg:
            for j in range(ICH):
                pltpu.make_async_copy(ones_v, deg_sh.at[dstb.at[j]],
                                      dsem).wait()
        return carry

    lax.fori_loop(0, nrows // ICH, chunk_body, 0)
    plsc.subcore_barrier()

    pltpu.sync_copy(acc_sh.at[pl.ds(s * ZCH, ZCH)],
                    agg_out.at[c, pl.ds(s * ZCH, ZCH)])
    if with_deg:
        pltpu.sync_copy(deg_sh.at[pl.ds(s * ZCH, ZCH)],
                        deg_out.at[pl.ds(c * NP + s * ZCH, ZCH)])


def _make_sc_agg(with_deg):
    mesh = plsc.VectorSubcoreMesh(core_axis_name="c", subcore_axis_name="s",
                                  num_cores=NC, num_subcores=NS)
    out_type = [jax.ShapeDtypeStruct((NC, NP, D), jnp.float32)]
    scratch_types = [
        pltpu.VMEM_SHARED((NP, D), jnp.float32),
        pltpu.VMEM((ICH, ROW), jnp.int32),
        pltpu.VMEM((ICH, ROW), jnp.int32),
        pltpu.VMEM((ROW, D), jnp.float32),
        pltpu.VMEM((ROW, D), jnp.float32),
        pltpu.SemaphoreType.DMA,
        pltpu.SemaphoreType.DMA,
    ]
    if with_deg:
        out_type.append(jax.ShapeDtypeStruct((NC * NP,), jnp.float32))
        scratch_types = [
            pltpu.VMEM_SHARED((NP, D), jnp.float32),
            pltpu.VMEM_SHARED((NP,), jnp.float32),
            pltpu.VMEM((ICH, ROW), jnp.int32),
            pltpu.VMEM((ICH, ROW), jnp.int32),
            pltpu.VMEM((ROW, D), jnp.float32),
            pltpu.VMEM((ROW, D), jnp.float32),
            pltpu.VMEM((ROW,), jnp.float32),
            pltpu.SemaphoreType.DMA,
            pltpu.SemaphoreType.DMA,
            pltpu.SemaphoreType.DMA,
        ]

        def body(m, src, dst, zacc, zdeg, agg_out, deg_out,
                 acc_sh, deg_sh, srcb, dstb, rows0, rows1, ones_v,
                 gsem0, gsem1, dsem):
            _sc_agg_body(True, m, src, dst, zacc, zdeg, agg_out, deg_out,
                         acc_sh, deg_sh, srcb, dstb, rows0, rows1, ones_v,
                         gsem0, gsem1, dsem)
    else:
        def body(m, src, dst, zacc, agg_out,
                 acc_sh, srcb, dstb, rows0, rows1, gsem0, gsem1):
            _sc_agg_body(False, m, src, dst, zacc, agg_out,
                         acc_sh, srcb, dstb, rows0, rows1, gsem0, gsem1)

    return pl.kernel(body, out_type=out_type, mesh=mesh,
                     scratch_types=scratch_types)


def _tc_in(x_ref, g_ref, b_ref, wi_ref, bi_ref, w1_ref, b1_ref,
           v1_ref, m1_ref):
    x = x_ref[...]
    mu = jnp.mean(x, axis=1, keepdims=True)
    xc = x - mu
    var = jnp.mean(xc * xc, axis=1, keepdims=True)
    xn = xc * lax.rsqrt(var + 1e-5) * g_ref[...] + b_ref[...]
    v1 = jnp.maximum(
        jnp.dot(xn, wi_ref[...], preferred_element_type=jnp.float32)
        + bi_ref[...], 0.0)
    v1_ref[...] = v1
    m1_ref[...] = (jnp.dot(v1, w1_ref[...], preferred_element_type=jnp.float32)
                   + b1_ref[...])


def _tc_mid(v1_ref, m1_ref, agg_ref, deg_ref, w2_ref, b2_ref,
            v2_ref, m2_ref, dinv_ref):
    a = agg_ref[0] + agg_ref[1]
    m1 = m1_ref[...]
    dinv = 1.0 / (deg_ref[0] + deg_ref[1] + 1.0)
    out = jnp.maximum((a + m1) * dinv, 0.0)
    v2 = BETA * v1_ref[...] + (1.0 - BETA) * out
    v2_ref[...] = v2
    m2_ref[...] = (jnp.dot(v2, w2_ref[...], preferred_element_type=jnp.float32)
                   + b2_ref[...])
    dinv_ref[...] = dinv


def _tc_out(v2_ref, m2_ref, agg_ref, dinv_ref, t_ref, s_ref):
    a = agg_ref[0] + agg_ref[1] + m2_ref[...]
    out = jnp.maximum(a * dinv_ref[...], 0.0)
    t2 = BETA * v2_ref[...] + (1.0 - BETA) * out
    nsq = jnp.sum(t2 * t2, axis=1, keepdims=True)
    n = jnp.maximum(jnp.sqrt(nsq), 1e-7)
    en = jnp.exp(n)
    einv = 1.0 / en
    t_ref[...] = 0.5 * (en + einv)
    s_ref[...] = (0.5 * (en - einv) / n) * t2


def _row_spec(b, d):
    return pl.BlockSpec((b, d), lambda i: (i, 0))


def _full_spec(shape):
    nd = len(shape)
    return pl.BlockSpec(shape, lambda i: (0,) * nd)


def _agg_spec(dw):
    return pl.BlockSpec((NC, BLK, dw), lambda i: (0, i, 0))


def _deg_spec():
    return pl.BlockSpec((NC, BLK, 1), lambda i: (0, i, 0))


def kernel(x, edge_index, ln_g, ln_b, W_in, b_in, W1, b1, W2, b2):
    src = edge_index[0].astype(jnp.int32)
    dst = edge_index[1].astype(jnp.int32)
    e = src.shape[0]
    align = NW * ROW * ICH
    ep = ((e + align - 1) // align) * align
    pad = ep - e
    src2d = jnp.concatenate(
        [src, jnp.zeros((pad,), jnp.int32)]).reshape(ep // ROW, ROW)
    dst_pad = N + (jnp.arange(pad, dtype=jnp.int32) % (NP - N))
    dst2d = jnp.concatenate([dst, dst_pad]).reshape(ep // ROW, ROW)
    zacc = jnp.zeros((NP, D), jnp.float32)
    zdeg = jnp.zeros((NP,), jnp.float32)

    g2 = ln_g.reshape(1, D)
    bn2 = ln_b.reshape(1, D)
    bi2 = b_in.reshape(1, D)
    b12 = b1.reshape(1, D)
    b22 = b2.reshape(1, D)

    v1, m1a = pl.pallas_call(
        _tc_in,
        grid=(GRID,),
        in_specs=[_row_spec(BLK, D), _full_spec((1, D)), _full_spec((1, D)),
                  _full_spec((D, D)), _full_spec((1, D)),
                  _full_spec((D, D)), _full_spec((1, D))],
        out_specs=[_row_spec(BLK, D), _row_spec(BLK, D)],
        out_shape=[jax.ShapeDtypeStruct((N, D), jnp.float32),
                   jax.ShapeDtypeStruct((N, D), jnp.float32)],
    )(x, g2, bn2, W_in, bi2, W1, b12)

    agg1, degp = _make_sc_agg(True)(m1a, src2d, dst2d, zacc, zdeg)
    degp = degp.reshape(NC, NP, 1)

    v2, m2, dinv = pl.pallas_call(
        _tc_mid,
        grid=(GRID,),
        in_specs=[_row_spec(BLK, D), _row_spec(BLK, D),
                  _agg_spec(D), _deg_spec(),
                  _full_spec((D, D)), _full_spec((1, D))],
        out_specs=[_row_spec(BLK, D), _row_spec(BLK, D), _row_spec(BLK, 1)],
        out_shape=[jax.ShapeDtypeStruct((N, D), jnp.float32),
                   jax.ShapeDtypeStruct((N, D), jnp.float32),
                   jax.ShapeDtypeStruct((N, 1), jnp.float32)],
    )(v1, m1a, agg1, degp, W2, b22)

    (agg2,) = _make_sc_agg(False)(m2, src2d, dst2d, zacc)

    t, sp = pl.pallas_call(
        _tc_out,
        grid=(GRID,),
        in_specs=[_row_spec(BLK, D), _row_spec(BLK, D),
                  _agg_spec(D), _row_spec(BLK, 1)],
        out_specs=[_row_spec(BLK, 1), _row_spec(BLK, D)],
        out_shape=[jax.ShapeDtypeStruct((N, 1), jnp.float32),
                   jax.ShapeDtypeStruct((N, D), jnp.float32)],
    )(v2, m2, agg2, dinv)

    return jnp.concatenate([t, sp], axis=-1)


# speedup vs baseline: 3.0183x; 3.0183x over previous
import jax
import jax.numpy as jnp
from jax import lax
from jax.experimental import pallas as pl
from jax.experimental.pallas import tpu as pltpu
from jax.experimental.pallas import tpu_sc as plsc

N = 10000
D = 128
BETA = 0.5
NC = 2
NS = 16
NW = NC * NS
ROW = 128
ICH = 8
NP = 10240
ZCH = NP // NS
BLK = 1000
GRID = N // BLK


def _sc_agg_body(with_deg, m_hbm, src_hbm, dst_hbm, zacc_hbm, *rest):
    if with_deg:
        (zdeg_hbm, agg_out, deg_out, acc_sh, deg_sh, srcb, dstb,
         rows0, rows1, ones_v, gsem0, gsem1, dsem) = rest
    else:
        zdeg_hbm = deg_out = deg_sh = ones_v = dsem = None
        (agg_out, acc_sh, srcb, dstb, rows0, rows1, gsem0, gsem1) = rest

    c = lax.axis_index("c")
    s = lax.axis_index("s")
    w = (1 - c) * NS + s
    nrows = src_hbm.shape[0] // NW
    base = w * nrows

    pltpu.sync_copy(zacc_hbm.at[pl.ds(s * ZCH, ZCH)],
                    acc_sh.at[pl.ds(s * ZCH, ZCH)])
    if with_deg:
        pltpu.sync_copy(zdeg_hbm.at[pl.ds(s * ZCH, ZCH)],
                        deg_sh.at[pl.ds(s * ZCH, ZCH)])
        for k in range(ROW // 16):
            ones_v[pl.ds(k * 16, 16)] = jnp.ones((16,), jnp.float32)
    plsc.subcore_barrier()

    def chunk_body(ci, carry):
        rb = base + ci * ICH
        pltpu.sync_copy(src_hbm.at[pl.ds(rb, ICH)], srcb)
        pltpu.sync_copy(dst_hbm.at[pl.ds(rb, ICH)], dstb)
        pltpu.async_copy(m_hbm.at[srcb.at[0]], rows0, gsem0)
        pltpu.async_copy(m_hbm.at[srcb.at[1]], rows1, gsem1)
        for j in range(ICH):
            buf, sem = (rows0, gsem0) if j % 2 == 0 else (rows1, gsem1)
            pltpu.make_async_copy(m_hbm.at[srcb.at[j]], buf, sem).wait()
            pltpu.sync_copy(buf, acc_sh.at[dstb.at[j]], add=True)
            if with_deg:
                pltpu.async_copy(ones_v, deg_sh.at[dstb.at[j]], dsem,
                                 add=True)
            if j + 2 < ICH:
                pltpu.async_copy(m_hbm.at[srcb.at[j + 2]], buf, sem)
        if with_deg:
            for j in range(ICH):
                pltpu.make_async_copy(ones_v, deg_sh.at[dstb.at[j]],
                                      dsem).wait()
        return carry

    lax.fori_loop(0, nrows // ICH, chunk_body, 0)
    plsc.subcore_barrier()

    pltpu.sync_copy(acc_sh.at[pl.ds(s * ZCH, ZCH)],
                    agg_out.at[c, pl.ds(s * ZCH, ZCH)])
    if with_deg:
        pltpu.sync_copy(deg_sh.at[pl.ds(s * ZCH, ZCH)],
                        deg_out.at[pl.ds(c * NP + s * ZCH, ZCH)])


def _make_sc_agg(with_deg):
    mesh = plsc.VectorSubcoreMesh(core_axis_name="c", subcore_axis_name="s",
                                  num_cores=NC, num_subcores=NS)
    out_type = [jax.ShapeDtypeStruct((NC, NP, D), jnp.float32)]
    scratch_types = [
        pltpu.VMEM_SHARED((NP, D), jnp.float32),
        pltpu.VMEM((ICH, ROW), jnp.int32),
        pltpu.VMEM((ICH, ROW), jnp.int32),
        pltpu.VMEM((ROW, D), jnp.float32),
        pltpu.VMEM((ROW, D), jnp.float32),
        pltpu.SemaphoreType.DMA,
        pltpu.SemaphoreType.DMA,
    ]
    if with_deg:
        out_type.append(jax.ShapeDtypeStruct((NC * NP,), jnp.float32))
        scratch_types = [
            pltpu.VMEM_SHARED((NP, D), jnp.float32),
            pltpu.VMEM_SHARED((NP,), jnp.float32),
            pltpu.VMEM((ICH, ROW), jnp.int32),
            pltpu.VMEM((ICH, ROW), jnp.int32),
            pltpu.VMEM((ROW, D), jnp.float32),
            pltpu.VMEM((ROW, D), jnp.float32),
            pltpu.VMEM((ROW,), jnp.float32),
            pltpu.SemaphoreType.DMA,
            pltpu.SemaphoreType.DMA,
            pltpu.SemaphoreType.DMA,
        ]

        def body(m, src, dst, zacc, zdeg, agg_out, deg_out,
                 acc_sh, deg_sh, srcb, dstb, rows0, rows1, ones_v,
                 gsem0, gsem1, dsem):
            _sc_agg_body(True, m, src, dst, zacc, zdeg, agg_out, deg_out,
                         acc_sh, deg_sh, srcb, dstb, rows0, rows1, ones_v,
                         gsem0, gsem1, dsem)
    else:
        def body(m, src, dst, zacc, agg_out,
                 acc_sh, srcb, dstb, rows0, rows1, gsem0, gsem1):
            _sc_agg_body(False, m, src, dst, zacc, agg_out,
                         acc_sh, srcb, dstb, rows0, rows1, gsem0, gsem1)

    return pl.kernel(body, out_type=out_type, mesh=mesh,
                     scratch_types=scratch_types)


def _tc_in(x_ref, g_ref, b_ref, wi_ref, bi_ref, w1_ref, b1_ref,
           v1_ref, m1_ref):
    x = x_ref[...]
    mu = jnp.mean(x, axis=1, keepdims=True)
    xc = x - mu
    var = jnp.mean(xc * xc, axis=1, keepdims=True)
    xn = xc * lax.rsqrt(var + 1e-5) * g_ref[...] + b_ref[...]
    v1 = jnp.maximum(
        jnp.dot(xn, wi_ref[...], preferred_element_type=jnp.float32)
        + bi_ref[...], 0.0)
    v1_ref[...] = v1
    m1_ref[...] = (jnp.dot(v1, w1_ref[...], preferred_element_type=jnp.float32)
                   + b1_ref[...])


def _tc_mid(v1_ref, m1_ref, agg_ref, deg_ref, w2_ref, b2_ref,
            v2_ref, m2_ref, dinv_ref):
    a = agg_ref[0] + agg_ref[1]
    m1 = m1_ref[...]
    dinv = 1.0 / (deg_ref[0] + deg_ref[1] + 1.0)
    out = jnp.maximum((a + m1) * dinv, 0.0)
    v2 = BETA * v1_ref[...] + (1.0 - BETA) * out
    v2_ref[...] = v2
    m2_ref[...] = (jnp.dot(v2, w2_ref[...], preferred_element_type=jnp.float32)
                   + b2_ref[...])
    dinv_ref[...] = dinv


def _tc_out(v2_ref, m2_ref, agg_ref, dinv_ref, t_ref, s_ref):
    a = agg_ref[0] + agg_ref[1] + m2_ref[...]
    out = jnp.maximum(a * dinv_ref[...], 0.0)
    t2 = BETA * v2_ref[...] + (1.0 - BETA) * out
    nsq = jnp.sum(t2 * t2, axis=1, keepdims=True)
    n = jnp.maximum(jnp.sqrt(nsq), 1e-7)
    en = jnp.exp(n)
    einv = 1.0 / en
    t_ref[...] = 0.5 * (en + einv)
    s_ref[...] = (0.5 * (en - einv) / n) * t2


def _row_spec(b, d):
    return pl.BlockSpec((b, d), lambda i: (i, 0))


def _full_spec(shape):
    nd = len(shape)
    return pl.BlockSpec(shape, lambda i: (0,) * nd)


def _agg_spec(dw):
    return pl.BlockSpec((NC, BLK, dw), lambda i: (0, i, 0))


def _deg_spec():
    return pl.BlockSpec((NC, BLK, 1), lambda i: (0, i, 0))


def kernel(x, edge_index, ln_g, ln_b, W_in, b_in, W1, b1, W2, b2):
    src = edge_index[0].astype(jnp.int32)
    dst = edge_index[1].astype(jnp.int32)
    e = src.shape[0]
    align = NW * ROW * ICH
    ep = ((e + align - 1) // align) * align
    pad = ep - e
    src_pad = jnp.arange(pad, dtype=jnp.int32) % N
    dst_pad = N + (jnp.arange(pad, dtype=jnp.int32) % (NP - N))
    src2d = jnp.concatenate([src, src_pad]).reshape(ep // ROW, ROW)
    dst2d = jnp.concatenate([dst, dst_pad]).reshape(ep // ROW, ROW)
    zacc = jnp.zeros((NP, D), jnp.float32)
    zdeg = jnp.zeros((NP,), jnp.float32)

    g2 = ln_g.reshape(1, D)
    bn2 = ln_b.reshape(1, D)
    bi2 = b_in.reshape(1, D)
    b12 = b1.reshape(1, D)
    b22 = b2.reshape(1, D)

    v1, m1a = pl.pallas_call(
        _tc_in,
        grid=(GRID,),
        in_specs=[_row_spec(BLK, D), _full_spec((1, D)), _full_spec((1, D)),
                  _full_spec((D, D)), _full_spec((1, D)),
                  _full_spec((D, D)), _full_spec((1, D))],
        out_specs=[_row_spec(BLK, D), _row_spec(BLK, D)],
        out_shape=[jax.ShapeDtypeStruct((N, D), jnp.float32),
                   jax.ShapeDtypeStruct((N, D), jnp.float32)],
    )(x, g2, bn2, W_in, bi2, W1, b12)

    agg1, degp = _make_sc_agg(True)(m1a, src2d, dst2d, zacc, zdeg)
    degp = degp.reshape(NC, NP, 1)

    v2, m2, dinv = pl.pallas_call(
        _tc_mid,
        grid=(GRID,),
        in_specs=[_row_spec(BLK, D), _row_spec(BLK, D),
                  _agg_spec(D), _deg_spec(),
                  _full_spec((D, D)), _full_spec((1, D))],
        out_specs=[_row_spec(BLK, D), _row_spec(BLK, D), _row_spec(BLK, 1)],
        out_shape=[jax.ShapeDtypeStruct((N, D), jnp.float32),
                   jax.ShapeDtypeStruct((N, D), jnp.float32),
                   jax.ShapeDtypeStruct((N, 1), jnp.float32)],
    )(v1, m1a, agg1, degp, W2, b22)

    (agg2,) = _make_sc_agg(False)(m2, src2d, dst2d, zacc)

    t, sp = pl.pallas_call(
        _tc_out,
        grid=(GRID,),
        in_specs=[_row_spec(BLK, D), _row_spec(BLK, D),
                  _agg_spec(D), _row_spec(BLK, 1)],
        out_specs=[_row_spec(BLK, 1), _row_spec(BLK, D)],
        out_shape=[jax.ShapeDtypeStruct((N, 1), jnp.float32),
                   jax.ShapeDtypeStruct((N, D), jnp.float32)],
    )(v2, m2, agg2, dinv)

    return jnp.concatenate([t, sp], axis=-1)


# speedup vs baseline: 3.3026x; 1.0942x over previous
import jax
import jax.numpy as jnp
from jax import lax
from jax.experimental import pallas as pl
from jax.experimental.pallas import tpu as pltpu
from jax.experimental.pallas import tpu_sc as plsc

N = 10000
D = 128
BETA = 0.5
NC = 2
NS = 16
NW = NC * NS
ROW = 128
ICH = 8
NP = 10240
ZCH = NP // NS
BLK = 1000
GRID = N // BLK


def _sc_agg_body(with_deg, m_hbm, src_hbm, dst_hbm, zacc_hbm, *rest):
    if with_deg:
        (zdeg_hbm, agg_out, deg_out, acc_sh, deg_sh, srcb, dstb,
         srcb2, dstb2, rows0, rows1, ones_v,
         gsem0, gsem1, isem0, isem1, dsem) = rest
    else:
        zdeg_hbm = deg_out = deg_sh = ones_v = dsem = None
        (agg_out, acc_sh, srcb, dstb, srcb2, dstb2, rows0, rows1,
         gsem0, gsem1, isem0, isem1) = rest

    c = lax.axis_index("c")
    s = lax.axis_index("s")
    w = (1 - c) * NS + s
    nrows = src_hbm.shape[0] // NW
    base = w * nrows

    pltpu.sync_copy(zacc_hbm.at[pl.ds(s * ZCH, ZCH)],
                    acc_sh.at[pl.ds(s * ZCH, ZCH)])
    if with_deg:
        pltpu.sync_copy(zdeg_hbm.at[pl.ds(s * ZCH, ZCH)],
                        deg_sh.at[pl.ds(s * ZCH, ZCH)])
        for k in range(ROW // 16):
            ones_v[pl.ds(k * 16, 16)] = jnp.ones((16,), jnp.float32)
    plsc.subcore_barrier()

    nchunks = nrows // ICH

    def idx_load(ci, sb, db, isem):
        rb = base + ci * ICH
        pltpu.async_copy(src_hbm.at[pl.ds(rb, ICH)], sb, isem)
        pltpu.async_copy(dst_hbm.at[pl.ds(rb, ICH)], db, isem)

    def idx_wait(ci, sb, db, isem):
        rb = base + ci * ICH
        pltpu.make_async_copy(src_hbm.at[pl.ds(rb, ICH)], sb, isem).wait()
        pltpu.make_async_copy(dst_hbm.at[pl.ds(rb, ICH)], db, isem).wait()

    def chunk_steps(sb, db):
        pltpu.async_copy(m_hbm.at[sb.at[0]], rows0, gsem0)
        pltpu.async_copy(m_hbm.at[sb.at[1]], rows1, gsem1)

        def istep(jj, carry):
            for b in range(2):
                j = 2 * jj + b
                buf, sem = (rows0, gsem0) if b == 0 else (rows1, gsem1)
                pltpu.make_async_copy(m_hbm.at[sb.at[j]], buf, sem).wait()
                pltpu.sync_copy(buf, acc_sh.at[db.at[j]], add=True)
                if with_deg:
                    pltpu.async_copy(ones_v, deg_sh.at[db.at[j]], dsem,
                                     add=True)

                @pl.when(j + 2 < ICH)
                def _():
                    pltpu.async_copy(m_hbm.at[sb.at[j + 2]], buf, sem)
            return carry

        lax.fori_loop(0, ICH // 2, istep, 0)
        if with_deg:
            for j in range(ICH):
                pltpu.make_async_copy(ones_v, deg_sh.at[db.at[j]],
                                      dsem).wait()

    idx_load(0, srcb, dstb, isem0)
    idx_load(1, srcb2, dstb2, isem1)
    idx_wait(0, srcb, dstb, isem0)

    def pair_body(i, carry):
        c0 = 2 * i

        @pl.when(i > 0)
        def _():
            idx_wait(c0, srcb, dstb, isem0)

        chunk_steps(srcb, dstb)

        @pl.when(c0 + 2 < nchunks)
        def _():
            idx_load(c0 + 2, srcb, dstb, isem0)

        idx_wait(c0 + 1, srcb2, dstb2, isem1)
        chunk_steps(srcb2, dstb2)

        @pl.when(c0 + 3 < nchunks)
        def _():
            idx_load(c0 + 3, srcb2, dstb2, isem1)

        return carry

    lax.fori_loop(0, nchunks // 2, pair_body, 0)
    plsc.subcore_barrier()

    pltpu.sync_copy(acc_sh.at[pl.ds(s * ZCH, ZCH)],
                    agg_out.at[c, pl.ds(s * ZCH, ZCH)])
    if with_deg:
        pltpu.sync_copy(deg_sh.at[pl.ds(s * ZCH, ZCH)],
                        deg_out.at[pl.ds(c * NP + s * ZCH, ZCH)])


def _make_sc_agg(with_deg):
    mesh = plsc.VectorSubcoreMesh(core_axis_name="c", subcore_axis_name="s",
                                  num_cores=NC, num_subcores=NS)
    out_type = [jax.ShapeDtypeStruct((NC, NP, D), jnp.float32)]
    scratch_types = [
        pltpu.VMEM_SHARED((NP, D), jnp.float32),
        pltpu.VMEM((ICH, ROW), jnp.int32),
        pltpu.VMEM((ICH, ROW), jnp.int32),
        pltpu.VMEM((ICH, ROW), jnp.int32),
        pltpu.VMEM((ICH, ROW), jnp.int32),
        pltpu.VMEM((ROW, D), jnp.float32),
        pltpu.VMEM((ROW, D), jnp.float32),
        pltpu.SemaphoreType.DMA,
        pltpu.SemaphoreType.DMA,
        pltpu.SemaphoreType.DMA,
        pltpu.SemaphoreType.DMA,
    ]
    if with_deg:
        out_type.append(jax.ShapeDtypeStruct((NC * NP,), jnp.float32))
        scratch_types = ([scratch_types[0],
                          pltpu.VMEM_SHARED((NP,), jnp.float32)]
                         + scratch_types[1:7]
                         + [pltpu.VMEM((ROW,), jnp.float32)]
                         + scratch_types[7:]
                         + [pltpu.SemaphoreType.DMA])

        def body(m, src, dst, zacc, zdeg, agg_out, deg_out,
                 acc_sh, deg_sh, srcb, dstb, srcb2, dstb2, rows0, rows1,
                 ones_v, gsem0, gsem1, isem0, isem1, dsem):
            _sc_agg_body(True, m, src, dst, zacc, zdeg, agg_out, deg_out,
                         acc_sh, deg_sh, srcb, dstb, srcb2, dstb2,
                         rows0, rows1, ones_v, gsem0, gsem1, isem0, isem1,
                         dsem)
    else:
        def body(m, src, dst, zacc, agg_out,
                 acc_sh, srcb, dstb, srcb2, dstb2, rows0, rows1,
                 gsem0, gsem1, isem0, isem1):
            _sc_agg_body(False, m, src, dst, zacc, agg_out,
                         acc_sh, srcb, dstb, srcb2, dstb2, rows0, rows1,
                         gsem0, gsem1, isem0, isem1)

    return pl.kernel(body, out_type=out_type, mesh=mesh,
                     scratch_types=scratch_types)


def _tc_in(x_ref, g_ref, b_ref, wi_ref, bi_ref, w1_ref, b1_ref,
           v1_ref, m1_ref):
    x = x_ref[...]
    mu = jnp.mean(x, axis=1, keepdims=True)
    xc = x - mu
    var = jnp.mean(xc * xc, axis=1, keepdims=True)
    xn = xc * lax.rsqrt(var + 1e-5) * g_ref[...] + b_ref[...]
    v1 = jnp.maximum(
        jnp.dot(xn, wi_ref[...], preferred_element_type=jnp.float32)
        + bi_ref[...], 0.0)
    v1_ref[...] = v1
    m1_ref[...] = (jnp.dot(v1, w1_ref[...], preferred_element_type=jnp.float32)
                   + b1_ref[...])


def _tc_mid(v1_ref, m1_ref, agg_ref, deg_ref, w2_ref, b2_ref,
            v2_ref, m2_ref, dinv_ref):
    a = agg_ref[0] + agg_ref[1]
    m1 = m1_ref[...]
    dinv = 1.0 / (deg_ref[0] + deg_ref[1] + 1.0)
    out = jnp.maximum((a + m1) * dinv, 0.0)
    v2 = BETA * v1_ref[...] + (1.0 - BETA) * out
    v2_ref[...] = v2
    m2_ref[...] = (jnp.dot(v2, w2_ref[...], preferred_element_type=jnp.float32)
                   + b2_ref[...])
    dinv_ref[...] = dinv


def _tc_out(v2_ref, m2_ref, agg_ref, dinv_ref, o_ref):
    a = agg_ref[0] + agg_ref[1] + m2_ref[...]
    out = jnp.maximum(a * dinv_ref[...], 0.0)
    t2 = BETA * v2_ref[...] + (1.0 - BETA) * out
    nsq = jnp.sum(t2 * t2, axis=1, keepdims=True)
    n = jnp.maximum(jnp.sqrt(nsq), 1e-7)
    en = jnp.exp(n)
    einv = 1.0 / en
    o_ref[...] = jnp.concatenate(
        [0.5 * (en + einv), (0.5 * (en - einv) / n) * t2], axis=1)


def _row_spec(b, d):
    return pl.BlockSpec((b, d), lambda i: (i, 0))


def _full_spec(shape):
    nd = len(shape)
    return pl.BlockSpec(shape, lambda i: (0,) * nd)


def _agg_spec(dw):
    return pl.BlockSpec((NC, BLK, dw), lambda i: (0, i, 0))


def _deg_spec():
    return pl.BlockSpec((NC, BLK, 1), lambda i: (0, i, 0))


def kernel(x, edge_index, ln_g, ln_b, W_in, b_in, W1, b1, W2, b2):
    src = edge_index[0].astype(jnp.int32)
    dst = edge_index[1].astype(jnp.int32)
    e = src.shape[0]
    align = NW * ROW * ICH
    ep = ((e + align - 1) // align) * align
    pad = ep - e
    src_pad = jnp.arange(pad, dtype=jnp.int32) % N
    dst_pad = N + (jnp.arange(pad, dtype=jnp.int32) % (NP - N))
    src2d = jnp.concatenate([src, src_pad]).reshape(ep // ROW, ROW)
    dst2d = jnp.concatenate([dst, dst_pad]).reshape(ep // ROW, ROW)
    zacc = jnp.zeros((NP, D), jnp.float32)
    zdeg = jnp.zeros((NP,), jnp.float32)

    g2 = ln_g.reshape(1, D)
    bn2 = ln_b.reshape(1, D)
    bi2 = b_in.reshape(1, D)
    b12 = b1.reshape(1, D)
    b22 = b2.reshape(1, D)

    v1, m1a = pl.pallas_call(
        _tc_in,
        grid=(GRID,),
        in_specs=[_row_spec(BLK, D), _full_spec((1, D)), _full_spec((1, D)),
                  _full_spec((D, D)), _full_spec((1, D)),
                  _full_spec((D, D)), _full_spec((1, D))],
        out_specs=[_row_spec(BLK, D), _row_spec(BLK, D)],
        out_shape=[jax.ShapeDtypeStruct((N, D), jnp.float32),
                   jax.ShapeDtypeStruct((N, D), jnp.float32)],
    )(x, g2, bn2, W_in, bi2, W1, b12)

    agg1, degp = _make_sc_agg(True)(m1a, src2d, dst2d, zacc, zdeg)
    degp = degp.reshape(NC, NP, 1)

    v2, m2, dinv = pl.pallas_call(
        _tc_mid,
        grid=(GRID,),
        in_specs=[_row_spec(BLK, D), _row_spec(BLK, D),
                  _agg_spec(D), _deg_spec(),
                  _full_spec((D, D)), _full_spec((1, D))],
        out_specs=[_row_spec(BLK, D), _row_spec(BLK, D), _row_spec(BLK, 1)],
        out_shape=[jax.ShapeDtypeStruct((N, D), jnp.float32),
                   jax.ShapeDtypeStruct((N, D), jnp.float32),
                   jax.ShapeDtypeStruct((N, 1), jnp.float32)],
    )(v1, m1a, agg1, degp, W2, b22)

    (agg2,) = _make_sc_agg(False)(m2, src2d, dst2d, zacc)

    return pl.pallas_call(
        _tc_out,
        grid=(GRID,),
        in_specs=[_row_spec(BLK, D), _row_spec(BLK, D),
                  _agg_spec(D), _row_spec(BLK, 1)],
        out_specs=_row_spec(BLK, D + 1),
        out_shape=jax.ShapeDtypeStruct((N, D + 1), jnp.float32),
    )(v2, m2, agg2, dinv)


# speedup vs baseline: 3.4367x; 1.0406x over previous
import jax
import jax.numpy as jnp
from jax import lax
from jax.experimental import pallas as pl
from jax.experimental.pallas import tpu as pltpu
from jax.experimental.pallas import tpu_sc as plsc

N = 10000
D = 128
BETA = 0.5
NC = 2
NS = 16
NW = NC * NS
ROW = 128
ICH = 8
NP = 10240
ZCH = NP // NS
BLK = 1000
GRID = N // BLK
DBLK = 1024


def _sc_agg_body(with_deg, m_hbm, src_hbm, dst_hbm, zacc_hbm, *rest):
    if with_deg:
        (zdeg_hbm, agg_out, deg_out, acc_sh, deg_sh, srcb, dstb,
         srcb2, dstb2, rows0, rows1, ones_v,
         gsem0, gsem1, isem0, isem1, dsem) = rest
    else:
        zdeg_hbm = deg_out = deg_sh = ones_v = dsem = None
        (agg_out, acc_sh, srcb, dstb, srcb2, dstb2, rows0, rows1,
         gsem0, gsem1, isem0, isem1) = rest

    c = lax.axis_index("c")
    s = lax.axis_index("s")
    w = (1 - c) * NS + s
    nrows = src_hbm.shape[0] // NW
    base = w * nrows

    pltpu.sync_copy(zacc_hbm.at[pl.ds(s * ZCH, ZCH)],
                    acc_sh.at[pl.ds(s * ZCH, ZCH)])
    if with_deg:
        pltpu.sync_copy(zdeg_hbm.at[pl.ds(s * ZCH, ZCH)],
                        deg_sh.at[pl.ds(s * ZCH, ZCH)])
        for k in range(ROW // 16):
            ones_v[pl.ds(k * 16, 16)] = jnp.ones((16,), jnp.float32)
    plsc.subcore_barrier()

    nchunks = nrows // ICH

    def idx_load(ci, sb, db, isem):
        rb = base + ci * ICH
        pltpu.async_copy(src_hbm.at[pl.ds(rb, ICH)], sb, isem)
        pltpu.async_copy(dst_hbm.at[pl.ds(rb, ICH)], db, isem)

    def idx_wait(ci, sb, db, isem):
        rb = base + ci * ICH
        pltpu.make_async_copy(src_hbm.at[pl.ds(rb, ICH)], sb, isem).wait()
        pltpu.make_async_copy(dst_hbm.at[pl.ds(rb, ICH)], db, isem).wait()

    def chunk_steps(sb, db):
        pltpu.async_copy(m_hbm.at[sb.at[0]], rows0, gsem0)
        pltpu.async_copy(m_hbm.at[sb.at[1]], rows1, gsem1)

        def istep(jj, carry):
            for b in range(2):
                j = 2 * jj + b
                buf, sem = (rows0, gsem0) if b == 0 else (rows1, gsem1)
                pltpu.make_async_copy(m_hbm.at[sb.at[j]], buf, sem).wait()
                pltpu.sync_copy(buf, acc_sh.at[db.at[j]], add=True)
                if with_deg:
                    pltpu.async_copy(ones_v, deg_sh.at[db.at[j]], dsem,
                                     add=True)

                @pl.when(j + 2 < ICH)
                def _():
                    pltpu.async_copy(m_hbm.at[sb.at[j + 2]], buf, sem)
            return carry

        lax.fori_loop(0, ICH // 2, istep, 0)
        if with_deg:
            for j in range(ICH):
                pltpu.make_async_copy(ones_v, deg_sh.at[db.at[j]],
                                      dsem).wait()

    idx_load(0, srcb, dstb, isem0)
    idx_load(1, srcb2, dstb2, isem1)
    idx_wait(0, srcb, dstb, isem0)

    def pair_body(i, carry):
        c0 = 2 * i

        @pl.when(i > 0)
        def _():
            idx_wait(c0, srcb, dstb, isem0)

        chunk_steps(srcb, dstb)

        @pl.when(c0 + 2 < nchunks)
        def _():
            idx_load(c0 + 2, srcb, dstb, isem0)

        idx_wait(c0 + 1, srcb2, dstb2, isem1)
        chunk_steps(srcb2, dstb2)

        @pl.when(c0 + 3 < nchunks)
        def _():
            idx_load(c0 + 3, srcb2, dstb2, isem1)

        return carry

    lax.fori_loop(0, nchunks // 2, pair_body, 0)
    plsc.subcore_barrier()

    pltpu.sync_copy(acc_sh.at[pl.ds(s * ZCH, ZCH)],
                    agg_out.at[c, pl.ds(s * ZCH, ZCH)])
    if with_deg:
        pltpu.sync_copy(deg_sh.at[pl.ds(s * ZCH, ZCH)],
                        deg_out.at[pl.ds(c * NP + s * ZCH, ZCH)])


def _make_sc_agg(with_deg):
    mesh = plsc.VectorSubcoreMesh(core_axis_name="c", subcore_axis_name="s",
                                  num_cores=NC, num_subcores=NS)
    out_type = [jax.ShapeDtypeStruct((NC, NP, D), jnp.float32)]
    scratch_types = [
        pltpu.VMEM_SHARED((NP, D), jnp.float32),
        pltpu.VMEM((ICH, ROW), jnp.int32),
        pltpu.VMEM((ICH, ROW), jnp.int32),
        pltpu.VMEM((ICH, ROW), jnp.int32),
        pltpu.VMEM((ICH, ROW), jnp.int32),
        pltpu.VMEM((ROW, D), jnp.float32),
        pltpu.VMEM((ROW, D), jnp.float32),
        pltpu.SemaphoreType.DMA,
        pltpu.SemaphoreType.DMA,
        pltpu.SemaphoreType.DMA,
        pltpu.SemaphoreType.DMA,
    ]
    if with_deg:
        out_type.append(jax.ShapeDtypeStruct((NC * NP,), jnp.float32))
        scratch_types = ([scratch_types[0],
                          pltpu.VMEM_SHARED((NP,), jnp.float32)]
                         + scratch_types[1:7]
                         + [pltpu.VMEM((ROW,), jnp.float32)]
                         + scratch_types[7:]
                         + [pltpu.SemaphoreType.DMA])

        def body(m, src, dst, zacc, zdeg, agg_out, deg_out,
                 acc_sh, deg_sh, srcb, dstb, srcb2, dstb2, rows0, rows1,
                 ones_v, gsem0, gsem1, isem0, isem1, dsem):
            _sc_agg_body(True, m, src, dst, zacc, zdeg, agg_out, deg_out,
                         acc_sh, deg_sh, srcb, dstb, srcb2, dstb2,
                         rows0, rows1, ones_v, gsem0, gsem1, isem0, isem1,
                         dsem)
    else:
        def body(m, src, dst, zacc, agg_out,
                 acc_sh, srcb, dstb, srcb2, dstb2, rows0, rows1,
                 gsem0, gsem1, isem0, isem1):
            _sc_agg_body(False, m, src, dst, zacc, agg_out,
                         acc_sh, srcb, dstb, srcb2, dstb2, rows0, rows1,
                         gsem0, gsem1, isem0, isem1)

    return pl.kernel(body, out_type=out_type, mesh=mesh,
                     scratch_types=scratch_types)


def _tc_in(x_ref, g_ref, b_ref, wi_ref, bi_ref, w1_ref, b1_ref,
           v1_ref, m1_ref):
    x = x_ref[...]
    mu = jnp.mean(x, axis=1, keepdims=True)
    xc = x - mu
    var = jnp.mean(xc * xc, axis=1, keepdims=True)
    xn = xc * lax.rsqrt(var + 1e-5) * g_ref[...] + b_ref[...]
    v1 = jnp.maximum(
        jnp.dot(xn, wi_ref[...], preferred_element_type=jnp.float32)
        + bi_ref[...], 0.0)
    v1_ref[...] = v1
    m1_ref[...] = (jnp.dot(v1, w1_ref[...], preferred_element_type=jnp.float32)
                   + b1_ref[...])


def _tc_deg(deg_ref, dinv_ref):
    d = deg_ref[0] + deg_ref[1] + 1.0
    dinv_ref[...] = (1.0 / d).reshape(DBLK, 1)


def _tc_edges(ei_ref, s_ref, d_ref):
    nfull = ei_ref.shape[1]
    npad = s_ref.shape[0] - nfull
    row = lax.broadcasted_iota(jnp.int32, (npad, ROW), 0)
    lane = lax.broadcasted_iota(jnp.int32, (npad, ROW), 1)
    flat = row * ROW + lane
    spad = flat % N
    dpad = N + flat % (NP - N)
    s_ref[...] = jnp.concatenate([ei_ref[0], spad], axis=0)
    d_ref[...] = jnp.concatenate([ei_ref[1], dpad], axis=0)


def _tc_mid(v1_ref, m1_ref, agg_ref, dinv_ref, w2_ref, b2_ref,
            v2_ref, m2_ref):
    a = agg_ref[0] + agg_ref[1]
    m1 = m1_ref[...]
    dinv = dinv_ref[...]
    out = jnp.maximum((a + m1) * dinv, 0.0)
    v2 = BETA * v1_ref[...] + (1.0 - BETA) * out
    v2_ref[...] = v2
    m2_ref[...] = (jnp.dot(v2, w2_ref[...], preferred_element_type=jnp.float32)
                   + b2_ref[...])


def _tc_out(v2_ref, m2_ref, agg_ref, dinv_ref, o_ref):
    a = agg_ref[0] + agg_ref[1] + m2_ref[...]
    out = jnp.maximum(a * dinv_ref[...], 0.0)
    t2 = BETA * v2_ref[...] + (1.0 - BETA) * out
    nsq = jnp.sum(t2 * t2, axis=1, keepdims=True)
    n = jnp.maximum(jnp.sqrt(nsq), 1e-7)
    en = jnp.exp(n)
    einv = 1.0 / en
    o_ref[...] = jnp.concatenate(
        [0.5 * (en + einv), (0.5 * (en - einv) / n) * t2], axis=1)


def _row_spec(b, d):
    return pl.BlockSpec((b, d), lambda i: (i, 0))


def _full_spec(shape):
    nd = len(shape)
    return pl.BlockSpec(shape, lambda i: (0,) * nd)


def _agg_spec(dw):
    return pl.BlockSpec((NC, BLK, dw), lambda i: (0, i, 0))


def kernel(x, edge_index, ln_g, ln_b, W_in, b_in, W1, b1, W2, b2):
    e = edge_index.shape[1]
    align = NW * ROW * ICH
    ep = ((e + align - 1) // align) * align
    ei2 = edge_index.astype(jnp.int32).reshape(2, e // ROW, ROW)
    src2d, dst2d = pl.pallas_call(
        _tc_edges,
        grid=(1,),
        in_specs=[_full_spec((2, e // ROW, ROW))],
        out_specs=[_full_spec((ep // ROW, ROW)), _full_spec((ep // ROW, ROW))],
        out_shape=[jax.ShapeDtypeStruct((ep // ROW, ROW), jnp.int32),
                   jax.ShapeDtypeStruct((ep // ROW, ROW), jnp.int32)],
    )(ei2)
    zacc = jnp.zeros((NP, D), jnp.float32)
    zdeg = jnp.zeros((NP,), jnp.float32)

    g2 = ln_g.reshape(1, D)
    bn2 = ln_b.reshape(1, D)
    bi2 = b_in.reshape(1, D)
    b12 = b1.reshape(1, D)
    b22 = b2.reshape(1, D)

    v1, m1a = pl.pallas_call(
        _tc_in,
        grid=(GRID,),
        in_specs=[_row_spec(BLK, D), _full_spec((1, D)), _full_spec((1, D)),
                  _full_spec((D, D)), _full_spec((1, D)),
                  _full_spec((D, D)), _full_spec((1, D))],
        out_specs=[_row_spec(BLK, D), _row_spec(BLK, D)],
        out_shape=[jax.ShapeDtypeStruct((N, D), jnp.float32),
                   jax.ShapeDtypeStruct((N, D), jnp.float32)],
    )(x, g2, bn2, W_in, bi2, W1, b12)

    agg1, degp = _make_sc_agg(True)(m1a, src2d, dst2d, zacc, zdeg)

    dinv = pl.pallas_call(
        _tc_deg,
        grid=(NP // DBLK,),
        in_specs=[pl.BlockSpec((NC, DBLK), lambda i: (0, i))],
        out_specs=_row_spec(DBLK, 1),
        out_shape=jax.ShapeDtypeStruct((N, 1), jnp.float32),
    )(degp.reshape(NC, NP))

    v2, m2 = pl.pallas_call(
        _tc_mid,
        grid=(GRID,),
        in_specs=[_row_spec(BLK, D), _row_spec(BLK, D),
                  _agg_spec(D), _row_spec(BLK, 1),
                  _full_spec((D, D)), _full_spec((1, D))],
        out_specs=[_row_spec(BLK, D), _row_spec(BLK, D)],
        out_shape=[jax.ShapeDtypeStruct((N, D), jnp.float32),
                   jax.ShapeDtypeStruct((N, D), jnp.float32)],
    )(v1, m1a, agg1, dinv, W2, b22)

    (agg2,) = _make_sc_agg(False)(m2, src2d, dst2d, zacc)

    return pl.pallas_call(
        _tc_out,
        grid=(GRID,),
        in_specs=[_row_spec(BLK, D), _row_spec(BLK, D),
                  _agg_spec(D), _row_spec(BLK, 1)],
        out_specs=_row_spec(BLK, D + 1),
        out_shape=jax.ShapeDtypeStruct((N, D + 1), jnp.float32),
    )(v2, m2, agg2, dinv)


# speedup vs baseline: 3.6300x; 1.0562x over previous
import jax
import jax.numpy as jnp
from jax import lax
from jax.experimental import pallas as pl
from jax.experimental.pallas import tpu as pltpu
from jax.experimental.pallas import tpu_sc as plsc

N = 10000
D = 128
BETA = 0.5
NC = 2
NS = 16
NW = NC * NS
ROW = 128
ICH = 8
NP = 10240
ZCH = NP // NS
BLK = 1000
GRID = N // BLK
DBLK = 1024


def _sc_agg_body(with_deg, m_hbm, src_hbm, dst_hbm, zacc_hbm, *rest):
    if with_deg:
        (zdeg_hbm, agg_out, deg_out, acc_sh, deg_sh, srcb, dstb,
         srcb2, dstb2, rows0, rows1, ones_v,
         gsem0, gsem1, isem0, isem1, dsem) = rest
    else:
        zdeg_hbm = deg_out = deg_sh = ones_v = dsem = None
        (agg_out, acc_sh, srcb, dstb, srcb2, dstb2, rows0, rows1,
         gsem0, gsem1, isem0, isem1) = rest

    c = lax.axis_index("c")
    s = lax.axis_index("s")
    w = (1 - c) * NS + s
    nrows = src_hbm.shape[0] // NW
    base = w * nrows

    pltpu.sync_copy(zacc_hbm.at[pl.ds(s * ZCH, ZCH)],
                    acc_sh.at[pl.ds(s * ZCH, ZCH)])
    if with_deg:
        pltpu.sync_copy(zdeg_hbm.at[pl.ds(s * ZCH, ZCH)],
                        deg_sh.at[pl.ds(s * ZCH, ZCH)])
        for k in range(ROW // 16):
            ones_v[pl.ds(k * 16, 16)] = jnp.ones((16,), jnp.float32)
    plsc.subcore_barrier()

    nchunks = nrows // ICH

    def idx_load(ci, sb, db, isem):
        rb = base + ci * ICH
        pltpu.async_copy(src_hbm.at[pl.ds(rb, ICH)], sb, isem)
        pltpu.async_copy(dst_hbm.at[pl.ds(rb, ICH)], db, isem)

    def idx_wait(ci, sb, db, isem):
        rb = base + ci * ICH
        pltpu.make_async_copy(src_hbm.at[pl.ds(rb, ICH)], sb, isem).wait()
        pltpu.make_async_copy(dst_hbm.at[pl.ds(rb, ICH)], db, isem).wait()

    def chunk_steps(sb, db, nb, next_cond):
        def istep(jj, carry):
            for b in range(2):
                j = 2 * jj + b
                buf, sem = (rows0, gsem0) if b == 0 else (rows1, gsem1)
                pltpu.make_async_copy(m_hbm.at[sb.at[j]], buf, sem).wait()
                pltpu.sync_copy(buf, acc_sh.at[db.at[j]], add=True)
                if with_deg:
                    pltpu.async_copy(ones_v, deg_sh.at[db.at[j]], dsem,
                                     add=True)
                pltpu.async_copy(m_hbm.at[sb.at[j + 2]], buf, sem)
            return carry

        lax.fori_loop(0, ICH // 2 - 1, istep, 0)
        for j in (ICH - 2, ICH - 1):
            buf, sem = (rows0, gsem0) if j % 2 == 0 else (rows1, gsem1)
            pltpu.make_async_copy(m_hbm.at[sb.at[j]], buf, sem).wait()
            pltpu.sync_copy(buf, acc_sh.at[db.at[j]], add=True)
            if with_deg:
                pltpu.async_copy(ones_v, deg_sh.at[db.at[j]], dsem,
                                 add=True)

            @pl.when(next_cond)
            def _():
                pltpu.async_copy(m_hbm.at[nb.at[j - (ICH - 2)]], buf, sem)

        if with_deg:
            for j in range(ICH):
                pltpu.make_async_copy(ones_v, deg_sh.at[db.at[j]],
                                      dsem).wait()

    idx_load(0, srcb, dstb, isem0)
    idx_load(1, srcb2, dstb2, isem1)
    idx_wait(0, srcb, dstb, isem0)
    pltpu.async_copy(m_hbm.at[srcb.at[0]], rows0, gsem0)
    pltpu.async_copy(m_hbm.at[srcb.at[1]], rows1, gsem1)

    def pair_body(i, carry):
        c0 = 2 * i
        idx_wait(c0 + 1, srcb2, dstb2, isem1)
        chunk_steps(srcb, dstb, srcb2, True)
        more = c0 + 2 < nchunks

        @pl.when(more)
        def _():
            idx_load(c0 + 2, srcb, dstb, isem0)
            idx_wait(c0 + 2, srcb, dstb, isem0)

        chunk_steps(srcb2, dstb2, srcb, more)

        @pl.when(c0 + 3 < nchunks)
        def _():
            idx_load(c0 + 3, srcb2, dstb2, isem1)

        return carry

    lax.fori_loop(0, nchunks // 2, pair_body, 0)
    plsc.subcore_barrier()

    pltpu.sync_copy(acc_sh.at[pl.ds(s * ZCH, ZCH)],
                    agg_out.at[c, pl.ds(s * ZCH, ZCH)])
    if with_deg:
        pltpu.sync_copy(deg_sh.at[pl.ds(s * ZCH, ZCH)],
                        deg_out.at[pl.ds(c * NP + s * ZCH, ZCH)])


def _make_sc_agg(with_deg):
    mesh = plsc.VectorSubcoreMesh(core_axis_name="c", subcore_axis_name="s",
                                  num_cores=NC, num_subcores=NS)
    out_type = [jax.ShapeDtypeStruct((NC, NP, D), jnp.float32)]
    scratch_types = [
        pltpu.VMEM_SHARED((NP, D), jnp.float32),
        pltpu.VMEM((ICH, ROW), jnp.int32),
        pltpu.VMEM((ICH, ROW), jnp.int32),
        pltpu.VMEM((ICH, ROW), jnp.int32),
        pltpu.VMEM((ICH, ROW), jnp.int32),
        pltpu.VMEM((ROW, D), jnp.float32),
        pltpu.VMEM((ROW, D), jnp.float32),
        pltpu.SemaphoreType.DMA,
        pltpu.SemaphoreType.DMA,
        pltpu.SemaphoreType.DMA,
        pltpu.SemaphoreType.DMA,
    ]
    if with_deg:
        out_type.append(jax.ShapeDtypeStruct((NC * NP,), jnp.float32))
        scratch_types = ([scratch_types[0],
                          pltpu.VMEM_SHARED((NP,), jnp.float32)]
                         + scratch_types[1:7]
                         + [pltpu.VMEM((ROW,), jnp.float32)]
                         + scratch_types[7:]
                         + [pltpu.SemaphoreType.DMA])

        def body(m, src, dst, zacc, zdeg, agg_out, deg_out,
                 acc_sh, deg_sh, srcb, dstb, srcb2, dstb2, rows0, rows1,
                 ones_v, gsem0, gsem1, isem0, isem1, dsem):
            _sc_agg_body(True, m, src, dst, zacc, zdeg, agg_out, deg_out,
                         acc_sh, deg_sh, srcb, dstb, srcb2, dstb2,
                         rows0, rows1, ones_v, gsem0, gsem1, isem0, isem1,
                         dsem)
    else:
        def body(m, src, dst, zacc, agg_out,
                 acc_sh, srcb, dstb, srcb2, dstb2, rows0, rows1,
                 gsem0, gsem1, isem0, isem1):
            _sc_agg_body(False, m, src, dst, zacc, agg_out,
                         acc_sh, srcb, dstb, srcb2, dstb2, rows0, rows1,
                         gsem0, gsem1, isem0, isem1)

    return pl.kernel(body, out_type=out_type, mesh=mesh,
                     scratch_types=scratch_types)


def _tc_in(x_ref, g_ref, b_ref, wi_ref, bi_ref, w1_ref, b1_ref,
           v1_ref, m1_ref):
    x = x_ref[...]
    mu = jnp.mean(x, axis=1, keepdims=True)
    xc = x - mu
    var = jnp.mean(xc * xc, axis=1, keepdims=True)
    xn = xc * lax.rsqrt(var + 1e-5) * g_ref[...] + b_ref[...]
    v1 = jnp.maximum(
        jnp.dot(xn, wi_ref[...], preferred_element_type=jnp.float32)
        + bi_ref[...], 0.0)
    v1_ref[...] = v1
    m1_ref[...] = (jnp.dot(v1, w1_ref[...], preferred_element_type=jnp.float32)
                   + b1_ref[...])


def _tc_deg(deg_ref, dinv_ref):
    d = deg_ref[0] + deg_ref[1] + 1.0
    dinv_ref[...] = (1.0 / d).reshape(DBLK, 1)


def _tc_edges(ei_ref, s_ref, d_ref):
    nfull = ei_ref.shape[1]
    npad = s_ref.shape[0] - nfull
    row = lax.broadcasted_iota(jnp.int32, (npad, ROW), 0)
    lane = lax.broadcasted_iota(jnp.int32, (npad, ROW), 1)
    flat = row * ROW + lane
    spad = flat % N
    dpad = N + flat % (NP - N)
    s_ref[...] = jnp.concatenate([ei_ref[0], spad], axis=0)
    d_ref[...] = jnp.concatenate([ei_ref[1], dpad], axis=0)


def _tc_mid(v1_ref, m1_ref, agg_ref, dinv_ref, w2_ref, b2_ref,
            v2_ref, m2_ref):
    a = agg_ref[0] + agg_ref[1]
    m1 = m1_ref[...]
    dinv = dinv_ref[...]
    out = jnp.maximum((a + m1) * dinv, 0.0)
    v2 = BETA * v1_ref[...] + (1.0 - BETA) * out
    v2_ref[...] = v2
    m2_ref[...] = (jnp.dot(v2, w2_ref[...], preferred_element_type=jnp.float32)
                   + b2_ref[...])


def _tc_out(v2_ref, m2_ref, agg_ref, dinv_ref, o_ref):
    a = agg_ref[0] + agg_ref[1] + m2_ref[...]
    out = jnp.maximum(a * dinv_ref[...], 0.0)
    t2 = BETA * v2_ref[...] + (1.0 - BETA) * out
    nsq = jnp.sum(t2 * t2, axis=1, keepdims=True)
    n = jnp.maximum(jnp.sqrt(nsq), 1e-7)
    en = jnp.exp(n)
    einv = 1.0 / en
    o_ref[...] = jnp.concatenate(
        [0.5 * (en + einv), (0.5 * (en - einv) / n) * t2], axis=1)


def _row_spec(b, d):
    return pl.BlockSpec((b, d), lambda i: (i, 0))


def _full_spec(shape):
    nd = len(shape)
    return pl.BlockSpec(shape, lambda i: (0,) * nd)


def _agg_spec(dw):
    return pl.BlockSpec((NC, BLK, dw), lambda i: (0, i, 0))


def kernel(x, edge_index, ln_g, ln_b, W_in, b_in, W1, b1, W2, b2):
    e = edge_index.shape[1]
    align = NW * ROW * ICH
    ep = ((e + align - 1) // align) * align
    ei2 = edge_index.astype(jnp.int32).reshape(2, e // ROW, ROW)
    src2d, dst2d = pl.pallas_call(
        _tc_edges,
        grid=(1,),
        in_specs=[_full_spec((2, e // ROW, ROW))],
        out_specs=[_full_spec((ep // ROW, ROW)), _full_spec((ep // ROW, ROW))],
        out_shape=[jax.ShapeDtypeStruct((ep // ROW, ROW), jnp.int32),
                   jax.ShapeDtypeStruct((ep // ROW, ROW), jnp.int32)],
    )(ei2)
    zacc = jnp.zeros((NP, D), jnp.float32)
    zdeg = jnp.zeros((NP,), jnp.float32)

    g2 = ln_g.reshape(1, D)
    bn2 = ln_b.reshape(1, D)
    bi2 = b_in.reshape(1, D)
    b12 = b1.reshape(1, D)
    b22 = b2.reshape(1, D)

    v1, m1a = pl.pallas_call(
        _tc_in,
        grid=(GRID,),
        in_specs=[_row_spec(BLK, D), _full_spec((1, D)), _full_spec((1, D)),
                  _full_spec((D, D)), _full_spec((1, D)),
                  _full_spec((D, D)), _full_spec((1, D))],
        out_specs=[_row_spec(BLK, D), _row_spec(BLK, D)],
        out_shape=[jax.ShapeDtypeStruct((N, D), jnp.float32),
                   jax.ShapeDtypeStruct((N, D), jnp.float32)],
    )(x, g2, bn2, W_in, bi2, W1, b12)

    agg1, degp = _make_sc_agg(True)(m1a, src2d, dst2d, zacc, zdeg)

    dinv = pl.pallas_call(
        _tc_deg,
        grid=(NP // DBLK,),
        in_specs=[pl.BlockSpec((NC, DBLK), lambda i: (0, i))],
        out_specs=_row_spec(DBLK, 1),
        out_shape=jax.ShapeDtypeStruct((N, 1), jnp.float32),
    )(degp.reshape(NC, NP))

    v2, m2 = pl.pallas_call(
        _tc_mid,
        grid=(GRID,),
        in_specs=[_row_spec(BLK, D), _row_spec(BLK, D),
                  _agg_spec(D), _row_spec(BLK, 1),
                  _full_spec((D, D)), _full_spec((1, D))],
        out_specs=[_row_spec(BLK, D), _row_spec(BLK, D)],
        out_shape=[jax.ShapeDtypeStruct((N, D), jnp.float32),
                   jax.ShapeDtypeStruct((N, D), jnp.float32)],
    )(v1, m1a, agg1, dinv, W2, b22)

    (agg2,) = _make_sc_agg(False)(m2, src2d, dst2d, zacc)

    return pl.pallas_call(
        _tc_out,
        grid=(GRID,),
        in_specs=[_row_spec(BLK, D), _row_spec(BLK, D),
                  _agg_spec(D), _row_spec(BLK, 1)],
        out_specs=_row_spec(BLK, D + 1),
        out_shape=jax.ShapeDtypeStruct((N, D + 1), jnp.float32),
    )(v2, m2, agg2, dinv)


# speedup vs baseline: 3.6349x; 1.0013x over previous
import jax
import jax.numpy as jnp
from jax import lax
from jax.experimental import pallas as pl
from jax.experimental.pallas import tpu as pltpu
from jax.experimental.pallas import tpu_sc as plsc

N = 10000
D = 128
BETA = 0.5
NC = 2
NS = 16
NW = NC * NS
ROW = 128
ICH = 8
NP = 10240
ZCH = NP // NS
BLK = 1000
GRID = N // BLK
DBLK = 1024


def _sc_agg_body(with_deg, m_hbm, src_hbm, dst_hbm, zacc_hbm, *rest):
    if with_deg:
        (zdeg_hbm, agg_out, deg_out, acc_sh, deg_sh, srcb, dstb,
         srcb2, dstb2, rows0, rows1, ones_v,
         gsem0, gsem1, isem0, isem1, dsem) = rest
    else:
        zdeg_hbm = deg_out = deg_sh = ones_v = dsem = None
        (agg_out, acc_sh, srcb, dstb, srcb2, dstb2, rows0, rows1,
         gsem0, gsem1, isem0, isem1) = rest

    c = lax.axis_index("c")
    s = lax.axis_index("s")
    w = c * NS + s
    nrows = src_hbm.shape[0] // NW
    base = w * nrows

    pltpu.sync_copy(zacc_hbm.at[pl.ds(s * ZCH, ZCH)],
                    acc_sh.at[pl.ds(s * ZCH, ZCH)])
    if with_deg:
        pltpu.sync_copy(zdeg_hbm.at[pl.ds(s * ZCH, ZCH)],
                        deg_sh.at[pl.ds(s * ZCH, ZCH)])
        for k in range(ROW // 16):
            ones_v[pl.ds(k * 16, 16)] = jnp.ones((16,), jnp.float32)
    plsc.subcore_barrier()

    nchunks = nrows // ICH

    def idx_load(ci, sb, db, isem):
        rb = base + ci * ICH
        pltpu.async_copy(src_hbm.at[pl.ds(rb, ICH)], sb, isem)
        pltpu.async_copy(dst_hbm.at[pl.ds(rb, ICH)], db, isem)

    def idx_wait(ci, sb, db, isem):
        rb = base + ci * ICH
        pltpu.make_async_copy(src_hbm.at[pl.ds(rb, ICH)], sb, isem).wait()
        pltpu.make_async_copy(dst_hbm.at[pl.ds(rb, ICH)], db, isem).wait()

    def chunk_steps(sb, db, nb, next_cond):
        def istep(jj, carry):
            for b in range(2):
                j = 2 * jj + b
                buf, sem = (rows0, gsem0) if b == 0 else (rows1, gsem1)
                pltpu.make_async_copy(m_hbm.at[sb.at[j]], buf, sem).wait()
                pltpu.sync_copy(buf, acc_sh.at[db.at[j]], add=True)
                if with_deg:
                    pltpu.async_copy(ones_v, deg_sh.at[db.at[j]], dsem,
                                     add=True)
                pltpu.async_copy(m_hbm.at[sb.at[j + 2]], buf, sem)
            return carry

        lax.fori_loop(0, ICH // 2 - 1, istep, 0)
        for j in (ICH - 2, ICH - 1):
            buf, sem = (rows0, gsem0) if j % 2 == 0 else (rows1, gsem1)
            pltpu.make_async_copy(m_hbm.at[sb.at[j]], buf, sem).wait()
            pltpu.sync_copy(buf, acc_sh.at[db.at[j]], add=True)
            if with_deg:
                pltpu.async_copy(ones_v, deg_sh.at[db.at[j]], dsem,
                                 add=True)

            @pl.when(next_cond)
            def _():
                pltpu.async_copy(m_hbm.at[nb.at[j - (ICH - 2)]], buf, sem)

        if with_deg:
            for j in range(ICH):
                pltpu.make_async_copy(ones_v, deg_sh.at[db.at[j]],
                                      dsem).wait()

    idx_load(0, srcb, dstb, isem0)
    idx_load(1, srcb2, dstb2, isem1)
    idx_wait(0, srcb, dstb, isem0)
    pltpu.async_copy(m_hbm.at[srcb.at[0]], rows0, gsem0)
    pltpu.async_copy(m_hbm.at[srcb.at[1]], rows1, gsem1)

    def pair_body(i, carry):
        c0 = 2 * i
        idx_wait(c0 + 1, srcb2, dstb2, isem1)
        chunk_steps(srcb, dstb, srcb2, True)
        more = c0 + 2 < nchunks

        @pl.when(more)
        def _():
            idx_load(c0 + 2, srcb, dstb, isem0)
            idx_wait(c0 + 2, srcb, dstb, isem0)

        chunk_steps(srcb2, dstb2, srcb, more)

        @pl.when(c0 + 3 < nchunks)
        def _():
            idx_load(c0 + 3, srcb2, dstb2, isem1)

        return carry

    lax.fori_loop(0, nchunks // 2, pair_body, 0)
    plsc.subcore_barrier()

    pltpu.sync_copy(acc_sh.at[pl.ds(s * ZCH, ZCH)],
                    agg_out.at[c, pl.ds(s * ZCH, ZCH)])
    if with_deg:
        pltpu.sync_copy(deg_sh.at[pl.ds(s * ZCH, ZCH)],
                        deg_out.at[pl.ds(c * NP + s * ZCH, ZCH)])


def _make_sc_agg(with_deg):
    mesh = plsc.VectorSubcoreMesh(core_axis_name="c", subcore_axis_name="s",
                                  num_cores=NC, num_subcores=NS)
    out_type = [jax.ShapeDtypeStruct((NC, NP, D), jnp.float32)]
    scratch_types = [
        pltpu.VMEM_SHARED((NP, D), jnp.float32),
        pltpu.VMEM((ICH, ROW), jnp.int32),
        pltpu.VMEM((ICH, ROW), jnp.int32),
        pltpu.VMEM((ICH, ROW), jnp.int32),
        pltpu.VMEM((ICH, ROW), jnp.int32),
        pltpu.VMEM((ROW, D), jnp.float32),
        pltpu.VMEM((ROW, D), jnp.float32),
        pltpu.SemaphoreType.DMA,
        pltpu.SemaphoreType.DMA,
        pltpu.SemaphoreType.DMA,
        pltpu.SemaphoreType.DMA,
    ]
    if with_deg:
        out_type.append(jax.ShapeDtypeStruct((NC * NP,), jnp.float32))
        scratch_types = ([scratch_types[0],
                          pltpu.VMEM_SHARED((NP,), jnp.float32)]
                         + scratch_types[1:7]
                         + [pltpu.VMEM((ROW,), jnp.float32)]
                         + scratch_types[7:]
                         + [pltpu.SemaphoreType.DMA])

        def body(m, src, dst, zacc, zdeg, agg_out, deg_out,
                 acc_sh, deg_sh, srcb, dstb, srcb2, dstb2, rows0, rows1,
                 ones_v, gsem0, gsem1, isem0, isem1, dsem):
            _sc_agg_body(True, m, src, dst, zacc, zdeg, agg_out, deg_out,
                         acc_sh, deg_sh, srcb, dstb, srcb2, dstb2,
                         rows0, rows1, ones_v, gsem0, gsem1, isem0, isem1,
                         dsem)
    else:
        def body(m, src, dst, zacc, agg_out,
                 acc_sh, srcb, dstb, srcb2, dstb2, rows0, rows1,
                 gsem0, gsem1, isem0, isem1):
            _sc_agg_body(False, m, src, dst, zacc, agg_out,
                         acc_sh, srcb, dstb, srcb2, dstb2, rows0, rows1,
                         gsem0, gsem1, isem0, isem1)

    return pl.kernel(body, out_type=out_type, mesh=mesh,
                     scratch_types=scratch_types)


def _tc_in(x_ref, g_ref, b_ref, wi_ref, bi_ref, w1_ref, b1_ref,
           v1_ref, m1_ref):
    x = x_ref[...]
    mu = jnp.mean(x, axis=1, keepdims=True)
    xc = x - mu
    var = jnp.mean(xc * xc, axis=1, keepdims=True)
    xn = xc * lax.rsqrt(var + 1e-5) * g_ref[...] + b_ref[...]
    v1 = jnp.maximum(
        jnp.dot(xn, wi_ref[...], preferred_element_type=jnp.float32)
        + bi_ref[...], 0.0)
    v1_ref[...] = v1
    m1_ref[...] = (jnp.dot(v1, w1_ref[...], preferred_element_type=jnp.float32)
                   + b1_ref[...])


def _tc_deg(deg_ref, dinv_ref):
    d = deg_ref[0] + deg_ref[1] + 1.0
    dinv_ref[...] = (1.0 / d).reshape(DBLK, 1)


def _tc_edges(ei_ref, s_ref, d_ref):
    nfull = ei_ref.shape[1]
    npad = s_ref.shape[0] - nfull
    row = lax.broadcasted_iota(jnp.int32, (npad, ROW), 0)
    lane = lax.broadcasted_iota(jnp.int32, (npad, ROW), 1)
    flat = row * ROW + lane
    spad = flat % N
    dpad = N + flat % (NP - N)
    s_ref[...] = jnp.concatenate([ei_ref[0], spad], axis=0)
    d_ref[...] = jnp.concatenate([ei_ref[1], dpad], axis=0)


def _tc_mid(v1_ref, m1_ref, agg_ref, dinv_ref, w2_ref, b2_ref,
            v2_ref, m2_ref):
    a = agg_ref[0] + agg_ref[1]
    m1 = m1_ref[...]
    dinv = dinv_ref[...]
    out = jnp.maximum((a + m1) * dinv, 0.0)
    v2 = BETA * v1_ref[...] + (1.0 - BETA) * out
    v2_ref[...] = v2
    m2_ref[...] = (jnp.dot(v2, w2_ref[...], preferred_element_type=jnp.float32)
                   + b2_ref[...])


def _tc_out(v2_ref, m2_ref, agg_ref, dinv_ref, o_ref):
    a = agg_ref[0] + agg_ref[1] + m2_ref[...]
    out = jnp.maximum(a * dinv_ref[...], 0.0)
    t2 = BETA * v2_ref[...] + (1.0 - BETA) * out
    nsq = jnp.sum(t2 * t2, axis=1, keepdims=True)
    n = jnp.maximum(jnp.sqrt(nsq), 1e-7)
    en = jnp.exp(n)
    einv = 1.0 / en
    o_ref[...] = jnp.concatenate(
        [0.5 * (en + einv), (0.5 * (en - einv) / n) * t2], axis=1)


def _row_spec(b, d):
    return pl.BlockSpec((b, d), lambda i: (i, 0))


def _full_spec(shape):
    nd = len(shape)
    return pl.BlockSpec(shape, lambda i: (0,) * nd)


def _agg_spec(dw):
    return pl.BlockSpec((NC, BLK, dw), lambda i: (0, i, 0))


def kernel(x, edge_index, ln_g, ln_b, W_in, b_in, W1, b1, W2, b2):
    e = edge_index.shape[1]
    align = NW * ROW * ICH
    ep = ((e + align - 1) // align) * align
    ei2 = edge_index.astype(jnp.int32).reshape(2, e // ROW, ROW)
    src2d, dst2d = pl.pallas_call(
        _tc_edges,
        grid=(1,),
        in_specs=[_full_spec((2, e // ROW, ROW))],
        out_specs=[_full_spec((ep // ROW, ROW)), _full_spec((ep // ROW, ROW))],
        out_shape=[jax.ShapeDtypeStruct((ep // ROW, ROW), jnp.int32),
                   jax.ShapeDtypeStruct((ep // ROW, ROW), jnp.int32)],
    )(ei2)
    zacc = jnp.zeros((NP, D), jnp.float32)
    zdeg = jnp.zeros((NP,), jnp.float32)

    g2 = ln_g.reshape(1, D)
    bn2 = ln_b.reshape(1, D)
    bi2 = b_in.reshape(1, D)
    b12 = b1.reshape(1, D)
    b22 = b2.reshape(1, D)

    v1, m1a = pl.pallas_call(
        _tc_in,
        grid=(GRID,),
        in_specs=[_row_spec(BLK, D), _full_spec((1, D)), _full_spec((1, D)),
                  _full_spec((D, D)), _full_spec((1, D)),
                  _full_spec((D, D)), _full_spec((1, D))],
        out_specs=[_row_spec(BLK, D), _row_spec(BLK, D)],
        out_shape=[jax.ShapeDtypeStruct((N, D), jnp.float32),
                   jax.ShapeDtypeStruct((N, D), jnp.float32)],
    )(x, g2, bn2, W_in, bi2, W1, b12)

    agg1, degp = _make_sc_agg(True)(m1a, src2d, dst2d, zacc, zdeg)

    dinv = pl.pallas_call(
        _tc_deg,
        grid=(NP // DBLK,),
        in_specs=[pl.BlockSpec((NC, DBLK), lambda i: (0, i))],
        out_specs=_row_spec(DBLK, 1),
        out_shape=jax.ShapeDtypeStruct((N, 1), jnp.float32),
    )(degp.reshape(NC, NP))

    v2, m2 = pl.pallas_call(
        _tc_mid,
        grid=(GRID,),
        in_specs=[_row_spec(BLK, D), _row_spec(BLK, D),
                  _agg_spec(D), _row_spec(BLK, 1),
                  _full_spec((D, D)), _full_spec((1, D))],
        out_specs=[_row_spec(BLK, D), _row_spec(BLK, D)],
        out_shape=[jax.ShapeDtypeStruct((N, D), jnp.float32),
                   jax.ShapeDtypeStruct((N, D), jnp.float32)],
    )(v1, m1a, agg1, dinv, W2, b22)

    (agg2,) = _make_sc_agg(False)(m2, src2d, dst2d, zacc)

    return pl.pallas_call(
        _tc_out,
        grid=(GRID,),
        in_specs=[_row_spec(BLK, D), _row_spec(BLK, D),
                  _agg_spec(D), _row_spec(BLK, 1)],
        out_specs=_row_spec(BLK, D + 1),
        out_shape=jax.ShapeDtypeStruct((N, D + 1), jnp.float32),
    )(v2, m2, agg2, dinv)
